# Initial kernel scaffold; baseline (speedup 1.0000x reference)
import functools

import jax
import jax.numpy as jnp
from jax import lax
from jax.experimental import pallas as pl
from jax.experimental.pallas import tpu as pltpu
from jax.experimental.pallas import tpu_sc as plsc

_N = 10000
_E = 320000
_D = 128
_C = 47
_L = 4

_BN = 2000
_B = 400
_SUB = 80
_NSUB = _B // _SUB
_EPT = _E // 16
_NCHUNK = _EPT // _B
_NPT = _N // 16
_RD = 25
_HALF = _D // 2


def _tc_encode(x, w, b):
    def body(x_ref, w_ref, b_ref, h_ref, cm_ref):
        i = pl.program_id(0)
        h = jnp.dot(x_ref[...], w_ref[...],
                    preferred_element_type=jnp.float32,
                    precision=lax.Precision.HIGHEST) + b_ref[...]
        h_ref[...] = h
        m = jnp.max(jnp.maximum(h, 0.0), axis=0, keepdims=True) + 1e-7

        @pl.when(i == 0)
        def _():
            cm_ref[...] = m

        @pl.when(i > 0)
        def _():
            cm_ref[...] = jnp.maximum(cm_ref[...], m)

    return pl.pallas_call(
        body,
        grid=(_N // _BN,),
        in_specs=[
            pl.BlockSpec((_BN, _D), lambda i: (i, 0)),
            pl.BlockSpec((_D, _D), lambda i: (0, 0)),
            pl.BlockSpec((1, _D), lambda i: (0, 0)),
        ],
        out_specs=[
            pl.BlockSpec((_BN, _D), lambda i: (i, 0)),
            pl.BlockSpec((1, _D), lambda i: (0, 0)),
        ],
        out_shape=[
            jax.ShapeDtypeStruct((_N, _D), jnp.float32),
            jax.ShapeDtypeStruct((1, _D), jnp.float32),
        ],
    )(x, w, b)


def _tc_post(t, aggr2, w, b, g, be, hprev):
    residual = hprev is not None

    def body(*refs):
        if residual:
            t_ref, a_ref, w_ref, b_ref, g_ref, be_ref, hp_ref = refs[:7]
            h_ref, t2_ref, cm_ref = refs[7:]
        else:
            t_ref, a_ref, w_ref, b_ref, g_ref, be_ref = refs[:6]
            h_ref, t2_ref, cm_ref = refs[6:]
        i = pl.program_id(0)
        w_full = w_ref[...]
        a0 = a_ref[0][:, :_HALF]
        a1 = a_ref[1][:, :_HALF]
        dot = functools.partial(jnp.dot,
                                preferred_element_type=jnp.float32,
                                precision=lax.Precision.HIGHEST)
        hn = (dot(t_ref[...], w_full)
              + dot(a0, w_full[:_HALF, :])
              + dot(a1, w_full[_HALF:, :])
              + b_ref[...])
        if residual:
            hn = hn + hp_ref[...]
        h_ref[...] = hn
        mu = jnp.mean(hn, axis=1, keepdims=True)
        var = jnp.mean((hn - mu) ** 2, axis=1, keepdims=True)
        ln = (hn - mu) * lax.rsqrt(var + 1e-5) * g_ref[...] + be_ref[...]
        t2 = jnp.maximum(ln, 0.0)
        t2_ref[...] = t2
        m = jnp.max(t2, axis=0, keepdims=True) + 1e-7

        @pl.when(i == 0)
        def _():
            cm_ref[...] = m

        @pl.when(i > 0)
        def _():
            cm_ref[...] = jnp.maximum(cm_ref[...], m)

    in_specs = [
        pl.BlockSpec((_BN, _D), lambda i: (i, 0)),
        pl.BlockSpec((2, _BN, _D), lambda i: (0, i, 0)),
        pl.BlockSpec((_D, _D), lambda i: (0, 0)),
        pl.BlockSpec((1, _D), lambda i: (0, 0)),
        pl.BlockSpec((1, _D), lambda i: (0, 0)),
        pl.BlockSpec((1, _D), lambda i: (0, 0)),
    ]
    args = [t, aggr2, w, b, g, be]
    if residual:
        in_specs.append(pl.BlockSpec((_BN, _D), lambda i: (i, 0)))
        args.append(hprev)
    return pl.pallas_call(
        body,
        grid=(_N // _BN,),
        in_specs=in_specs,
        out_specs=[
            pl.BlockSpec((_BN, _D), lambda i: (i, 0)),
            pl.BlockSpec((_BN, _D), lambda i: (i, 0)),
            pl.BlockSpec((1, _D), lambda i: (0, 0)),
        ],
        out_shape=[
            jax.ShapeDtypeStruct((_N, _D), jnp.float32),
            jax.ShapeDtypeStruct((_N, _D), jnp.float32),
            jax.ShapeDtypeStruct((1, _D), jnp.float32),
        ],
    )(*args)


def _tc_predict(t, wp, bp):
    def body(t_ref, w_ref, b_ref, o_ref):
        lg = jnp.dot(t_ref[...], w_ref[...],
                     preferred_element_type=jnp.float32,
                     precision=lax.Precision.HIGHEST) + b_ref[...]
        mask = lax.broadcasted_iota(jnp.int32, (_BN, _D), 1) < _C
        mx = jnp.max(jnp.where(mask, lg, -jnp.inf), axis=1, keepdims=True)
        se = jnp.sum(jnp.where(mask, jnp.exp(lg - mx), 0.0),
                     axis=1, keepdims=True)
        o_ref[...] = lg - mx - jnp.log(se)

    return pl.pallas_call(
        body,
        grid=(_N // _BN,),
        in_specs=[
            pl.BlockSpec((_BN, _D), lambda i: (i, 0)),
            pl.BlockSpec((_D, _D), lambda i: (0, 0)),
            pl.BlockSpec((1, _D), lambda i: (0, 0)),
        ],
        out_specs=pl.BlockSpec((_BN, _D), lambda i: (i, 0)),
        out_shape=jax.ShapeDtypeStruct((_N, _D), jnp.float32),
    )(t, wp, bp)


def _sc_edge_pass(t, src2, dst2, cm):
    mesh = plsc.VectorSubcoreMesh(core_axis_name="c", subcore_axis_name="s")

    @functools.partial(
        pl.kernel,
        out_type=jax.ShapeDtypeStruct((2, _N, _D), jnp.float32),
        mesh=mesh,
        scratch_types=[
            pltpu.VMEM((_NSUB, _SUB), jnp.int32),
            pltpu.VMEM((_NSUB, _SUB), jnp.int32),
            pltpu.VMEM((_B, _D), jnp.float32),
            pltpu.VMEM((_B, _D), jnp.float32),
            pltpu.VMEM((_HALF,), jnp.float32),
            pltpu.VMEM((_RD, _D), jnp.float32),
            pltpu.VMEM((_RD, _D), jnp.float32),
            pltpu.VMEM_SHARED((_N, _D), jnp.float32),
            pltpu.SemaphoreType.DMA,
        ],
    )
    def k(t_hbm, src_hbm, dst_hbm, cm_hbm, out_hbm,
          src_v, dst_v, rows_v, vals_v, cm_v, dbuf, abuf, acc, sem):
        c = lax.axis_index("c")
        s = lax.axis_index("s")
        c64 = c * _HALF
        zv = jnp.zeros((16,), jnp.float32)

        def zrow(i, _):
            for j in range(_D // 16):
                rows_v[i, pl.ds(j * 16, 16)] = zv
            return 0
        lax.fori_loop(0, _B, zrow, 0)
        pltpu.sync_copy(rows_v, acc.at[pl.ds(s * _NPT, _B)])
        pltpu.sync_copy(rows_v.at[pl.ds(0, _NPT - _B)],
                        acc.at[pl.ds(s * _NPT + _B, _NPT - _B)])
        pltpu.sync_copy(cm_hbm.at[pl.ds(c64, _HALF)], cm_v)
        plsc.subcore_barrier()

        cms = [cm_v[pl.ds(j * 16, 16)] for j in range(_HALF // 16)]

        def chunk(g, _):
            r0 = s * (_EPT // _SUB) + g * _NSUB
            pltpu.sync_copy(src_hbm.at[pl.ds(r0, _NSUB)], src_v)
            pltpu.sync_copy(dst_hbm.at[pl.ds(r0, _NSUB)], dst_v)
            cps = [
                pltpu.async_copy(t_hbm.at[src_v.at[j]],
                                 rows_v.at[pl.ds(j * _SUB, _SUB)], sem)
                for j in range(_NSUB)
            ]
            for cp in cps:
                cp.wait()

            def row(i, _):
                for j in range(_HALF // 16):
                    r = rows_v[i, pl.ds(c64 + j * 16, 16)]
                    m = jnp.maximum(r, 0.0) + 1e-7
                    e = jnp.exp(m - cms[j])
                    vals_v[i, pl.ds(j * 16, 16)] = e
                    vals_v[i, pl.ds(_HALF + j * 16, 16)] = m * e
                return 0
            lax.fori_loop(0, _B, row, 0)

            for j in range(_NSUB):
                pltpu.sync_copy(vals_v.at[pl.ds(j * _SUB, _SUB)],
                                acc.at[dst_v.at[j]], add=True)
            return 0
        lax.fori_loop(0, _NCHUNK, chunk, 0)
        plsc.subcore_barrier()

        def dump(k2, _):
            r0 = s * _NPT + k2 * _RD
            pltpu.sync_copy(acc.at[pl.ds(r0, _RD)], dbuf)

            def drow(i, _):
                for j in range(_HALF // 16):
                    d = dbuf[i, pl.ds(j * 16, 16)]
                    nm = dbuf[i, pl.ds(_HALF + j * 16, 16)]
                    abuf[i, pl.ds(j * 16, 16)] = nm / (d + 1e-16)
                return 0
            lax.fori_loop(0, _RD, drow, 0)
            pltpu.sync_copy(abuf, out_hbm.at[c, pl.ds(r0, _RD)])
            return 0
        lax.fori_loop(0, _NPT // _RD, dump, 0)

    return k(t, src2, dst2, cm)


def kernel(x, edge_index, W_enc, b_enc, Wg, bg, gamma, beta, W_pred, b_pred):
    src2 = edge_index[0].reshape(_E // _SUB, _SUB)
    dst2 = edge_index[1].reshape(_E // _SUB, _SUB)
    b_enc2 = b_enc.reshape(1, _D)
    wp = jnp.zeros((_D, _D), jnp.float32).at[:, :_C].set(W_pred)
    bp = jnp.zeros((1, _D), jnp.float32).at[0, :_C].set(b_pred)

    t, cm = _tc_encode(x, W_enc, b_enc2)
    h = None
    for l in range(_L):
        aggr2 = _sc_edge_pass(t, src2, dst2, cm.reshape(_D))
        h, t, cm = _tc_post(t, aggr2, Wg[l], bg[l].reshape(1, _D),
                            gamma[l].reshape(1, _D), beta[l].reshape(1, _D),
                            h)
    full = _tc_predict(t, wp, bp)
    return full[:, :_C]


# speedup vs baseline: 2.5585x; 2.5585x over previous
import functools

import jax
import jax.numpy as jnp
from jax import lax
from jax.experimental import pallas as pl
from jax.experimental.pallas import tpu as pltpu
from jax.experimental.pallas import tpu_sc as plsc

_N = 10000
_E = 320000
_D = 128
_C = 47
_L = 4

_BN = 2000
_B = 200
_SUB = 40
_NSUB = _B // _SUB
_EPT = _E // 16
_NCHUNK = _EPT // _B
_NP = 10240
_NPT = _NP // 16
_RD = 16
_HALF = _D // 2


def _tc_encode(x, w, b):
    def body(x_ref, w_ref, b_ref, h_ref, cm_ref):
        i = pl.program_id(0)
        h = jnp.dot(x_ref[...], w_ref[...],
                    preferred_element_type=jnp.float32,
                    precision=lax.Precision.HIGHEST) + b_ref[...]
        h_ref[...] = h
        m = jnp.max(jnp.maximum(h, 0.0), axis=0, keepdims=True) + 1e-7

        @pl.when(i == 0)
        def _():
            cm_ref[...] = m

        @pl.when(i > 0)
        def _():
            cm_ref[...] = jnp.maximum(cm_ref[...], m)

    return pl.pallas_call(
        body,
        grid=(_N // _BN,),
        in_specs=[
            pl.BlockSpec((_BN, _D), lambda i: (i, 0)),
            pl.BlockSpec((_D, _D), lambda i: (0, 0)),
            pl.BlockSpec((1, _D), lambda i: (0, 0)),
        ],
        out_specs=[
            pl.BlockSpec((_BN, _D), lambda i: (i, 0)),
            pl.BlockSpec((1, _D), lambda i: (0, 0)),
        ],
        out_shape=[
            jax.ShapeDtypeStruct((_N, _D), jnp.float32),
            jax.ShapeDtypeStruct((1, _D), jnp.float32),
        ],
    )(x, w, b)


def _tc_post(t, aggr2, w, b, g, be, hprev):
    residual = hprev is not None

    def body(*refs):
        if residual:
            t_ref, a_ref, w_ref, b_ref, g_ref, be_ref, hp_ref = refs[:7]
            h_ref, t2_ref, cm_ref = refs[7:]
        else:
            t_ref, a_ref, w_ref, b_ref, g_ref, be_ref = refs[:6]
            h_ref, t2_ref, cm_ref = refs[6:]
        i = pl.program_id(0)
        w_full = w_ref[...]
        a0 = a_ref[0][:, :_HALF]
        a1 = a_ref[1][:, :_HALF]
        dot = functools.partial(jnp.dot,
                                preferred_element_type=jnp.float32,
                                precision=lax.Precision.HIGHEST)
        hn = (dot(t_ref[...], w_full)
              + dot(a0, w_full[:_HALF, :])
              + dot(a1, w_full[_HALF:, :])
              + b_ref[...])
        if residual:
            hn = hn + hp_ref[...]
        h_ref[...] = hn
        mu = jnp.mean(hn, axis=1, keepdims=True)
        var = jnp.mean((hn - mu) ** 2, axis=1, keepdims=True)
        ln = (hn - mu) * lax.rsqrt(var + 1e-5) * g_ref[...] + be_ref[...]
        t2 = jnp.maximum(ln, 0.0)
        t2_ref[...] = t2
        m = jnp.max(t2, axis=0, keepdims=True) + 1e-7

        @pl.when(i == 0)
        def _():
            cm_ref[...] = m

        @pl.when(i > 0)
        def _():
            cm_ref[...] = jnp.maximum(cm_ref[...], m)

    in_specs = [
        pl.BlockSpec((_BN, _D), lambda i: (i, 0)),
        pl.BlockSpec((2, _BN, _D), lambda i: (0, i, 0)),
        pl.BlockSpec((_D, _D), lambda i: (0, 0)),
        pl.BlockSpec((1, _D), lambda i: (0, 0)),
        pl.BlockSpec((1, _D), lambda i: (0, 0)),
        pl.BlockSpec((1, _D), lambda i: (0, 0)),
    ]
    args = [t, aggr2, w, b, g, be]
    if residual:
        in_specs.append(pl.BlockSpec((_BN, _D), lambda i: (i, 0)))
        args.append(hprev)
    return pl.pallas_call(
        body,
        grid=(_N // _BN,),
        in_specs=in_specs,
        out_specs=[
            pl.BlockSpec((_BN, _D), lambda i: (i, 0)),
            pl.BlockSpec((_BN, _D), lambda i: (i, 0)),
            pl.BlockSpec((1, _D), lambda i: (0, 0)),
        ],
        out_shape=[
            jax.ShapeDtypeStruct((_N, _D), jnp.float32),
            jax.ShapeDtypeStruct((_N, _D), jnp.float32),
            jax.ShapeDtypeStruct((1, _D), jnp.float32),
        ],
    )(*args)


def _tc_predict(t, wp, bp):
    def body(t_ref, w_ref, b_ref, o_ref):
        lg = jnp.dot(t_ref[...], w_ref[...],
                     preferred_element_type=jnp.float32,
                     precision=lax.Precision.HIGHEST) + b_ref[...]
        mask = lax.broadcasted_iota(jnp.int32, (_BN, _D), 1) < _C
        mx = jnp.max(jnp.where(mask, lg, -jnp.inf), axis=1, keepdims=True)
        se = jnp.sum(jnp.where(mask, jnp.exp(lg - mx), 0.0),
                     axis=1, keepdims=True)
        o_ref[...] = lg - mx - jnp.log(se)

    return pl.pallas_call(
        body,
        grid=(_N // _BN,),
        in_specs=[
            pl.BlockSpec((_BN, _D), lambda i: (i, 0)),
            pl.BlockSpec((_D, _D), lambda i: (0, 0)),
            pl.BlockSpec((1, _D), lambda i: (0, 0)),
        ],
        out_specs=pl.BlockSpec((_BN, _D), lambda i: (i, 0)),
        out_shape=jax.ShapeDtypeStruct((_N, _D), jnp.float32),
    )(t, wp, bp)


def _sc_edge_pass(t, src, dst, cm):
    mesh = plsc.VectorSubcoreMesh(core_axis_name="c", subcore_axis_name="s")

    @functools.partial(
        pl.kernel,
        out_type=jax.ShapeDtypeStruct((2, _NP, _D), jnp.float32),
        mesh=mesh,
        scratch_types=[
            pltpu.VMEM((_B,), jnp.int32),
            pltpu.VMEM((_NSUB, _SUB), jnp.int32),
            pltpu.VMEM((_B, _D), jnp.float32),
            pltpu.VMEM((_HALF,), jnp.float32),
            pltpu.VMEM((_RD, _D), jnp.float32),
            pltpu.VMEM((_RD, _D), jnp.float32),
            pltpu.VMEM_SHARED((_NP, _D), jnp.float32),
            pltpu.SemaphoreType.DMA,
        ],
    )
    def k(t_hbm, src_hbm, dst_hbm, cm_hbm, out_hbm,
          src_v, dst_v, rows_v, cm_v, dbuf, abuf, acc, sem):
        c = lax.axis_index("c")
        s = lax.axis_index("s")
        c64 = c * _HALF
        zv = jnp.zeros((16,), jnp.float32)

        def zrow(i, _):
            for j in range(_D // 16):
                rows_v[i, pl.ds(j * 16, 16)] = zv
            return 0
        lax.fori_loop(0, _B, zrow, 0)
        for z in range(_NPT // _B):
            pltpu.sync_copy(rows_v, acc.at[pl.ds(s * _NPT + z * _B, _B)])
        if _NPT % _B:
            pltpu.sync_copy(rows_v.at[pl.ds(0, _NPT % _B)],
                            acc.at[pl.ds(s * _NPT + (_NPT // _B) * _B,
                                         _NPT % _B)])
        pltpu.sync_copy(cm_hbm.at[pl.ds(c64, _HALF)], cm_v)
        plsc.subcore_barrier()

        cms = [cm_v[pl.ds(j * 16, 16)] for j in range(_HALF // 16)]

        def chunk(g, _):
            e0 = s * _EPT + g * _B
            pltpu.sync_copy(src_hbm.at[pl.ds(e0, _B)], src_v)
            for j in range(_NSUB):
                pltpu.sync_copy(dst_hbm.at[pl.ds(e0 + j * _SUB, _SUB)],
                                dst_v.at[j])
            cps = [
                pltpu.async_copy(t_hbm.at[src_v.at[pl.ds(j * _SUB, _SUB)]],
                                 rows_v.at[pl.ds(j * _SUB, _SUB)], sem)
                for j in range(_NSUB)
            ]
            for cp in cps:
                cp.wait()

            def row(i, _):
                for j in range(_HALF // 16):
                    r = rows_v[i, pl.ds(c64 + j * 16, 16)]
                    m = jnp.maximum(r, 0.0) + 1e-7
                    e = jnp.exp(m - cms[j])
                    rows_v[i, pl.ds(j * 16, 16)] = e
                    rows_v[i, pl.ds(_HALF + j * 16, 16)] = m * e
                return 0
            lax.fori_loop(0, _B, row, 0)

            for j in range(_NSUB):
                pltpu.sync_copy(rows_v.at[pl.ds(j * _SUB, _SUB)],
                                acc.at[dst_v.at[j]], add=True)
            return 0
        lax.fori_loop(0, _NCHUNK, chunk, 0)
        plsc.subcore_barrier()

        def dump(k2, _):
            r0 = s * _NPT + k2 * _RD
            pltpu.sync_copy(acc.at[pl.ds(r0, _RD)], dbuf)

            def drow(i, _):
                for j in range(_HALF // 16):
                    d = dbuf[i, pl.ds(j * 16, 16)]
                    nm = dbuf[i, pl.ds(_HALF + j * 16, 16)]
                    abuf[i, pl.ds(j * 16, 16)] = nm / (d + 1e-16)
                return 0
            lax.fori_loop(0, _RD, drow, 0)
            pltpu.sync_copy(abuf, out_hbm.at[c, pl.ds(r0, _RD)])
            return 0
        lax.fori_loop(0, _NPT // _RD, dump, 0)

    return k(t, src, dst, cm)


def kernel(x, edge_index, W_enc, b_enc, Wg, bg, gamma, beta, W_pred, b_pred):
    src = edge_index[0]
    dst = edge_index[1]
    b_enc2 = b_enc.reshape(1, _D)
    wp = jnp.zeros((_D, _D), jnp.float32).at[:, :_C].set(W_pred)
    bp = jnp.zeros((1, _D), jnp.float32).at[0, :_C].set(b_pred)

    t, cm = _tc_encode(x, W_enc, b_enc2)
    h = None
    for l in range(_L):
        aggr2 = _sc_edge_pass(t, src, dst, cm.reshape(_D))
        h, t, cm = _tc_post(t, aggr2, Wg[l], bg[l].reshape(1, _D),
                            gamma[l].reshape(1, _D), beta[l].reshape(1, _D),
                            h)
    full = _tc_predict(t, wp, bp)
    return full[:, :_C]


# speedup vs baseline: 2.8042x; 1.0960x over previous
import functools

import jax
import jax.numpy as jnp
from jax import lax
from jax.experimental import pallas as pl
from jax.experimental.pallas import tpu as pltpu
from jax.experimental.pallas import tpu_sc as plsc

_N = 10000
_E = 320000
_D = 128
_C = 47
_L = 4

_BN = 2000
_B = 160
_SUB = 40
_NSUB = _B // _SUB
_EPT = _E // 16
_NCHUNK = _EPT // _B
_NPAIR = (_NCHUNK + 1) // 2
_IROW = _E // _SUB
_IPAD = 16 * _NPAIR * 8 - _IROW
_NP = 10240
_NPT = _NP // 16
_RD = 8
_HALF = _D // 2


def _tc_encode(x, w, b):
    def body(x_ref, w_ref, b_ref, h_ref, cm_ref):
        i = pl.program_id(0)
        h = jnp.dot(x_ref[...], w_ref[...],
                    preferred_element_type=jnp.float32,
                    precision=lax.Precision.HIGHEST) + b_ref[...]
        h_ref[...] = h
        m = jnp.max(jnp.maximum(h, 0.0), axis=0, keepdims=True) + 1e-7

        @pl.when(i == 0)
        def _():
            cm_ref[...] = m

        @pl.when(i > 0)
        def _():
            cm_ref[...] = jnp.maximum(cm_ref[...], m)

    return pl.pallas_call(
        body,
        grid=(_N // _BN,),
        in_specs=[
            pl.BlockSpec((_BN, _D), lambda i: (i, 0)),
            pl.BlockSpec((_D, _D), lambda i: (0, 0)),
            pl.BlockSpec((1, _D), lambda i: (0, 0)),
        ],
        out_specs=[
            pl.BlockSpec((_BN, _D), lambda i: (i, 0)),
            pl.BlockSpec((1, _D), lambda i: (0, 0)),
        ],
        out_shape=[
            jax.ShapeDtypeStruct((_N, _D), jnp.float32),
            jax.ShapeDtypeStruct((1, _D), jnp.float32),
        ],
    )(x, w, b)


def _tc_post(t, aggr2, w, b, g, be, hprev):
    residual = hprev is not None

    def body(*refs):
        if residual:
            t_ref, a_ref, w_ref, b_ref, g_ref, be_ref, hp_ref = refs[:7]
            h_ref, t2_ref, cm_ref = refs[7:]
        else:
            t_ref, a_ref, w_ref, b_ref, g_ref, be_ref = refs[:6]
            h_ref, t2_ref, cm_ref = refs[6:]
        i = pl.program_id(0)
        w_full = w_ref[...]
        a0 = a_ref[0][:, :_HALF]
        a1 = a_ref[1][:, :_HALF]
        dot = functools.partial(jnp.dot,
                                preferred_element_type=jnp.float32,
                                precision=lax.Precision.HIGHEST)
        hn = (dot(t_ref[...], w_full)
              + dot(a0, w_full[:_HALF, :])
              + dot(a1, w_full[_HALF:, :])
              + b_ref[...])
        if residual:
            hn = hn + hp_ref[...]
        h_ref[...] = hn
        mu = jnp.mean(hn, axis=1, keepdims=True)
        var = jnp.mean((hn - mu) ** 2, axis=1, keepdims=True)
        ln = (hn - mu) * lax.rsqrt(var + 1e-5) * g_ref[...] + be_ref[...]
        t2 = jnp.maximum(ln, 0.0)
        t2_ref[...] = t2
        m = jnp.max(t2, axis=0, keepdims=True) + 1e-7

        @pl.when(i == 0)
        def _():
            cm_ref[...] = m

        @pl.when(i > 0)
        def _():
            cm_ref[...] = jnp.maximum(cm_ref[...], m)

    in_specs = [
        pl.BlockSpec((_BN, _D), lambda i: (i, 0)),
        pl.BlockSpec((2, _BN, _D), lambda i: (0, i, 0)),
        pl.BlockSpec((_D, _D), lambda i: (0, 0)),
        pl.BlockSpec((1, _D), lambda i: (0, 0)),
        pl.BlockSpec((1, _D), lambda i: (0, 0)),
        pl.BlockSpec((1, _D), lambda i: (0, 0)),
    ]
    args = [t, aggr2, w, b, g, be]
    if residual:
        in_specs.append(pl.BlockSpec((_BN, _D), lambda i: (i, 0)))
        args.append(hprev)
    return pl.pallas_call(
        body,
        grid=(_N // _BN,),
        in_specs=in_specs,
        out_specs=[
            pl.BlockSpec((_BN, _D), lambda i: (i, 0)),
            pl.BlockSpec((_BN, _D), lambda i: (i, 0)),
            pl.BlockSpec((1, _D), lambda i: (0, 0)),
        ],
        out_shape=[
            jax.ShapeDtypeStruct((_N, _D), jnp.float32),
            jax.ShapeDtypeStruct((_N, _D), jnp.float32),
            jax.ShapeDtypeStruct((1, _D), jnp.float32),
        ],
    )(*args)


def _tc_predict(t, wp, bp):
    def body(t_ref, w_ref, b_ref, o_ref):
        lg = jnp.dot(t_ref[...], w_ref[...],
                     preferred_element_type=jnp.float32,
                     precision=lax.Precision.HIGHEST) + b_ref[...]
        mask = lax.broadcasted_iota(jnp.int32, (_BN, _D), 1) < _C
        mx = jnp.max(jnp.where(mask, lg, -jnp.inf), axis=1, keepdims=True)
        se = jnp.sum(jnp.where(mask, jnp.exp(lg - mx), 0.0),
                     axis=1, keepdims=True)
        o_ref[...] = lg - mx - jnp.log(se)

    return pl.pallas_call(
        body,
        grid=(_N // _BN,),
        in_specs=[
            pl.BlockSpec((_BN, _D), lambda i: (i, 0)),
            pl.BlockSpec((_D, _D), lambda i: (0, 0)),
            pl.BlockSpec((1, _D), lambda i: (0, 0)),
        ],
        out_specs=pl.BlockSpec((_BN, _D), lambda i: (i, 0)),
        out_shape=jax.ShapeDtypeStruct((_N, _D), jnp.float32),
    )(t, wp, bp)


def _sc_edge_pass(t, src_r, dst_r, cm):
    mesh = plsc.VectorSubcoreMesh(core_axis_name="c", subcore_axis_name="s")

    @functools.partial(
        pl.kernel,
        out_type=jax.ShapeDtypeStruct((2, _NP, _D), jnp.float32),
        mesh=mesh,
        scratch_types=[
            pltpu.VMEM((2 * _NSUB, _SUB), jnp.int32),
            pltpu.VMEM((2 * _NSUB, _SUB), jnp.int32),
            pltpu.VMEM((2, _B, _D), jnp.float32),
            pltpu.VMEM((_HALF,), jnp.float32),
            pltpu.VMEM((_RD, _D), jnp.float32),
            pltpu.VMEM((2, _RD, _D), jnp.float32),
            pltpu.VMEM_SHARED((_NP, _D), jnp.float32),
            pltpu.SemaphoreType.DMA,
            pltpu.SemaphoreType.DMA,
            pltpu.SemaphoreType.DMA,
        ],
    )
    def k(t_hbm, src_hbm, dst_hbm, cm_hbm, out_hbm,
          srcg, dstg, rows_v, cm_v, dbuf, abuf, acc, gsem, ssem, wsem):
        c = lax.axis_index("c")
        s = lax.axis_index("s")
        c64 = c * _HALF
        zv = jnp.zeros((16,), jnp.float32)

        def zrow(i, _):
            for p in range(2):
                for j in range(_D // 16):
                    rows_v[p, i, pl.ds(j * 16, 16)] = zv
            return 0
        lax.fori_loop(0, _B, zrow, 0)
        for z in range(_NPT // _B):
            pltpu.sync_copy(rows_v.at[z % 2],
                            acc.at[pl.ds(s * _NPT + z * _B, _B)])
        pltpu.sync_copy(cm_hbm.at[pl.ds(c64, _HALF)], cm_v)
        plsc.subcore_barrier()

        cms = [cm_v[pl.ds(j * 16, 16)] for j in range(_HALF // 16)]

        def compute(p):
            def rowpair(i, _):
                for r2 in range(2):
                    ii = i * 2 + r2
                    for j in range(_HALF // 16):
                        r = rows_v[p, ii, pl.ds(c64 + j * 16, 16)]
                        m = jnp.maximum(r, 0.0) + 1e-7
                        e = jnp.exp(m - cms[j])
                        rows_v[p, ii, pl.ds(j * 16, 16)] = e
                        rows_v[p, ii, pl.ds(_HALF + j * 16, 16)] = m * e
                return 0
            lax.fori_loop(0, _B // 2, rowpair, 0)

        def pair(tt, _):
            gp = tt * 16 + s
            r0 = gp * 2 * _NSUB
            pltpu.sync_copy(src_hbm.at[pl.ds(r0, 2 * _NSUB)], srcg)
            pltpu.sync_copy(dst_hbm.at[pl.ds(r0, 2 * _NSUB)], dstg)
            gA = [pltpu.async_copy(
                      t_hbm.at[srcg.at[j]],
                      rows_v.at[0, pl.ds(j * _SUB, _SUB)], gsem)
                  for j in range(_NSUB)]
            gB = [pltpu.async_copy(
                      t_hbm.at[srcg.at[_NSUB + j]],
                      rows_v.at[1, pl.ds(j * _SUB, _SUB)], gsem)
                  for j in range(_NSUB)]
            for cp in gA:
                cp.wait()
            compute(0)

            @pl.when(gp < _IROW // (2 * _NSUB))
            def _():
                sA = [pltpu.async_copy(
                          rows_v.at[0, pl.ds(j * _SUB, _SUB)],
                          acc.at[dstg.at[j]], ssem, add=True)
                      for j in range(_NSUB)]
                for cp in gB:
                    cp.wait()
                compute(1)
                sB = [pltpu.async_copy(
                          rows_v.at[1, pl.ds(j * _SUB, _SUB)],
                          acc.at[dstg.at[_NSUB + j]], ssem, add=True)
                      for j in range(_NSUB)]
                for cp in sB:
                    cp.wait()
                for cp in sA:
                    cp.wait()

            @pl.when(gp >= _IROW // (2 * _NSUB))
            def _():
                for cp in gB:
                    cp.wait()
            return 0
        lax.fori_loop(0, _NPAIR, pair, 0)
        plsc.subcore_barrier()

        def dump(k2, _):
            r0 = s * _NPT + k2 * 2 * _RD
            ws = []
            for h in range(2):
                pltpu.sync_copy(acc.at[pl.ds(r0 + h * _RD, _RD)], dbuf)

                def drow(i, _):
                    for j in range(_HALF // 16):
                        d = dbuf[i, pl.ds(j * 16, 16)]
                        nm = dbuf[i, pl.ds(_HALF + j * 16, 16)]
                        abuf[h, i, pl.ds(j * 16, 16)] = nm / (d + 1e-16)
                    return 0
                lax.fori_loop(0, _RD, drow, 0)
                ws.append(pltpu.async_copy(
                    abuf.at[h], out_hbm.at[c, pl.ds(r0 + h * _RD, _RD)],
                    wsem))
            for cp in ws:
                cp.wait()
            return 0
        lax.fori_loop(0, _NPT // (2 * _RD), dump, 0)

    return k(t, src_r, dst_r, cm)


def kernel(x, edge_index, W_enc, b_enc, Wg, bg, gamma, beta, W_pred, b_pred):
    src_r = jnp.pad(edge_index[0].reshape(_IROW, _SUB), ((0, _IPAD), (0, 0)))
    dst_r = jnp.pad(edge_index[1].reshape(_IROW, _SUB), ((0, _IPAD), (0, 0)))
    b_enc2 = b_enc.reshape(1, _D)
    wp = jnp.zeros((_D, _D), jnp.float32).at[:, :_C].set(W_pred)
    bp = jnp.zeros((1, _D), jnp.float32).at[0, :_C].set(b_pred)

    t, cm = _tc_encode(x, W_enc, b_enc2)
    h = None
    for l in range(_L):
        aggr2 = _sc_edge_pass(t, src_r, dst_r, cm.reshape(_D))
        h, t, cm = _tc_post(t, aggr2, Wg[l], bg[l].reshape(1, _D),
                            gamma[l].reshape(1, _D), beta[l].reshape(1, _D),
                            h)
    full = _tc_predict(t, wp, bp)
    return full[:, :_C]


# speedup vs baseline: 13.5054x; 4.8162x over previous
import functools

import jax
import jax.numpy as jnp
from jax import lax
from jax.experimental import pallas as pl
from jax.experimental.pallas import tpu as pltpu
from jax.experimental.pallas import tpu_sc as plsc

_N = 10000
_E = 320000
_D = 128
_C = 47
_L = 4

_BN = 2000
_B = 80
_NBUF = 4
_EPT = _E // 16
_NCHUNK = _EPT // _B
_GRP = 8
_IROWT = _NCHUNK
_IPADT = 256
_NP = 10240
_NPT = _NP // 16
_RD = 8
_HALF = _D // 2


def _tc_encode(x, w, b):
    def body(x_ref, w_ref, b_ref, h_ref, cm_ref):
        i = pl.program_id(0)
        h = jnp.dot(x_ref[...], w_ref[...],
                    preferred_element_type=jnp.float32,
                    precision=lax.Precision.HIGHEST) + b_ref[...]
        h_ref[...] = h
        m = jnp.max(jnp.maximum(h, 0.0), axis=0, keepdims=True) + 1e-7

        @pl.when(i == 0)
        def _():
            cm_ref[...] = m

        @pl.when(i > 0)
        def _():
            cm_ref[...] = jnp.maximum(cm_ref[...], m)

    return pl.pallas_call(
        body,
        grid=(_N // _BN,),
        in_specs=[
            pl.BlockSpec((_BN, _D), lambda i: (i, 0)),
            pl.BlockSpec((_D, _D), lambda i: (0, 0)),
            pl.BlockSpec((1, _D), lambda i: (0, 0)),
        ],
        out_specs=[
            pl.BlockSpec((_BN, _D), lambda i: (i, 0)),
            pl.BlockSpec((1, _D), lambda i: (0, 0)),
        ],
        out_shape=[
            jax.ShapeDtypeStruct((_N, _D), jnp.float32),
            jax.ShapeDtypeStruct((1, _D), jnp.float32),
        ],
    )(x, w, b)


def _tc_post(t, aggr2, w, b, g, be, hprev):
    residual = hprev is not None

    def body(*refs):
        if residual:
            t_ref, a_ref, w_ref, b_ref, g_ref, be_ref, hp_ref = refs[:7]
            h_ref, t2_ref, cm_ref = refs[7:]
        else:
            t_ref, a_ref, w_ref, b_ref, g_ref, be_ref = refs[:6]
            h_ref, t2_ref, cm_ref = refs[6:]
        i = pl.program_id(0)
        w_full = w_ref[...]
        a0 = a_ref[0]
        a1 = a_ref[1]
        dot = functools.partial(jnp.dot,
                                preferred_element_type=jnp.float32,
                                precision=lax.Precision.HIGHEST)
        hn = (dot(t_ref[...], w_full)
              + dot(a0, w_full[:_HALF, :])
              + dot(a1, w_full[_HALF:, :])
              + b_ref[...])
        if residual:
            hn = hn + hp_ref[...]
        h_ref[...] = hn
        mu = jnp.mean(hn, axis=1, keepdims=True)
        var = jnp.mean((hn - mu) ** 2, axis=1, keepdims=True)
        ln = (hn - mu) * lax.rsqrt(var + 1e-5) * g_ref[...] + be_ref[...]
        t2 = jnp.maximum(ln, 0.0)
        t2_ref[...] = t2
        m = jnp.max(t2, axis=0, keepdims=True) + 1e-7

        @pl.when(i == 0)
        def _():
            cm_ref[...] = m

        @pl.when(i > 0)
        def _():
            cm_ref[...] = jnp.maximum(cm_ref[...], m)

    in_specs = [
        pl.BlockSpec((_BN, _D), lambda i: (i, 0)),
        pl.BlockSpec((2, _BN, _HALF), lambda i: (0, i, 0)),
        pl.BlockSpec((_D, _D), lambda i: (0, 0)),
        pl.BlockSpec((1, _D), lambda i: (0, 0)),
        pl.BlockSpec((1, _D), lambda i: (0, 0)),
        pl.BlockSpec((1, _D), lambda i: (0, 0)),
    ]
    args = [t, aggr2, w, b, g, be]
    if residual:
        in_specs.append(pl.BlockSpec((_BN, _D), lambda i: (i, 0)))
        args.append(hprev)
    return pl.pallas_call(
        body,
        grid=(_N // _BN,),
        in_specs=in_specs,
        out_specs=[
            pl.BlockSpec((_BN, _D), lambda i: (i, 0)),
            pl.BlockSpec((_BN, _D), lambda i: (i, 0)),
            pl.BlockSpec((1, _D), lambda i: (0, 0)),
        ],
        out_shape=[
            jax.ShapeDtypeStruct((_N, _D), jnp.float32),
            jax.ShapeDtypeStruct((_N, _D), jnp.float32),
            jax.ShapeDtypeStruct((1, _D), jnp.float32),
        ],
    )(*args)


def _tc_predict(t, wp, bp):
    def body(t_ref, w_ref, b_ref, o_ref):
        lg = jnp.dot(t_ref[...], w_ref[...],
                     preferred_element_type=jnp.float32,
                     precision=lax.Precision.HIGHEST) + b_ref[...]
        mask = lax.broadcasted_iota(jnp.int32, (_BN, _D), 1) < _C
        mx = jnp.max(jnp.where(mask, lg, -jnp.inf), axis=1, keepdims=True)
        se = jnp.sum(jnp.where(mask, jnp.exp(lg - mx), 0.0),
                     axis=1, keepdims=True)
        o_ref[...] = lg - mx - jnp.log(se)

    return pl.pallas_call(
        body,
        grid=(_N // _BN,),
        in_specs=[
            pl.BlockSpec((_BN, _D), lambda i: (i, 0)),
            pl.BlockSpec((_D, _D), lambda i: (0, 0)),
            pl.BlockSpec((1, _D), lambda i: (0, 0)),
        ],
        out_specs=pl.BlockSpec((_BN, _D), lambda i: (i, 0)),
        out_shape=jax.ShapeDtypeStruct((_N, _D), jnp.float32),
    )(t, wp, bp)


def _sc_edge_pass(t, src_r, dst_r, cm):
    mesh = plsc.VectorSubcoreMesh(core_axis_name="c", subcore_axis_name="s")

    @functools.partial(
        pl.kernel,
        out_type=jax.ShapeDtypeStruct((2, _NP, _HALF), jnp.float32),
        mesh=mesh,
        scratch_types=[
            pltpu.VMEM((2, _GRP, _B), jnp.int32),
            pltpu.VMEM((2, _GRP, _B), jnp.int32),
            pltpu.VMEM((_NBUF, _B, _D), jnp.float32),
            pltpu.VMEM((_HALF,), jnp.float32),
            pltpu.VMEM((_RD, _D), jnp.float32),
            pltpu.VMEM((2, _RD, _HALF), jnp.float32),
            pltpu.VMEM_SHARED((_NP, _D), jnp.float32),
            pltpu.SemaphoreType.DMA,
            pltpu.SemaphoreType.DMA,
            pltpu.SemaphoreType.DMA,
            pltpu.SemaphoreType.DMA,
            pltpu.SemaphoreType.DMA,
        ],
    )
    def k(t_hbm, src_hbm, dst_hbm, cm_hbm, out_hbm,
          srcg, dstg, rows_v, cm_v, dbuf, abuf, acc, sm0, sm1, sm2, sm3,
          wsem):
        sems = [sm0, sm1, sm2, sm3]
        c = lax.axis_index("c")
        s = lax.axis_index("s")
        c64 = c * _HALF
        zv = jnp.zeros((16,), jnp.float32)

        def zrow(i, _):
            for j in range(_D // 16):
                rows_v[0, i, pl.ds(j * 16, 16)] = zv
            return 0
        lax.fori_loop(0, _B, zrow, 0)
        for z in range(_NPT // _B):
            pltpu.sync_copy(rows_v.at[0],
                            acc.at[pl.ds(s * _NPT + z * _B, _B)])
        pltpu.sync_copy(cm_hbm.at[pl.ds(c64, _HALF)], cm_v)
        plsc.subcore_barrier()

        cms = [cm_v[pl.ds(j * 16, 16)] for j in range(_HALF // 16)]
        nj = _HALF // 16

        def compute(p):
            def load_row(ii):
                return [rows_v[p, ii, pl.ds(c64 + j * 16, 16)]
                        for j in range(nj)]

            def store_row(ii, rs):
                ms = [jnp.maximum(r, 0.0) + 1e-7 for r in rs]
                es = [jnp.exp(ms[j] - cms[j]) for j in range(nj)]
                for j in range(nj):
                    rows_v[p, ii, pl.ds(j * 16, 16)] = es[j]
                    rows_v[p, ii, pl.ds(_HALF + j * 16, 16)] = ms[j] * es[j]

            def body(i, rs):
                rs_next = load_row(i + 1)
                store_row(i, rs)
                return rs_next
            last = lax.fori_loop(0, _B - 1, body, load_row(0))
            store_row(_B - 1, last)

        def issue_gather(g, p):
            grp = lax.rem(lax.div(g, _GRP), 2)
            row = lax.rem(g, _GRP)
            return pltpu.async_copy(t_hbm.at[srcg.at[grp, row]],
                                    rows_v.at[p], sems[p])

        pltpu.sync_copy(src_hbm.at[s, pl.ds(0, _GRP)], srcg.at[0])
        pltpu.sync_copy(dst_hbm.at[s, pl.ds(0, _GRP)], dstg.at[0])
        issue_gather(0, 0)
        issue_gather(1, 1)

        def round_(r, _):
            for p in range(_NBUF):
                g = r * _NBUF + p
                q = (p + 2) % _NBUF

                @pl.when(g < _NCHUNK)
                def _():
                    pltpu.make_async_copy(t_hbm.at[srcg.at[0, 0]],
                                          rows_v.at[p], sems[p]).wait()
                    compute(p)
                    grp = lax.rem(lax.div(g, _GRP), 2)
                    row = lax.rem(g, _GRP)
                    pltpu.async_copy(rows_v.at[p],
                                     acc.at[dstg.at[grp, row]],
                                     sems[p], add=True)

                @pl.when((g >= 2) & (g < _NCHUNK + 2))
                def _():
                    pltpu.make_async_copy(rows_v.at[q],
                                          acc.at[dstg.at[0, 0]],
                                          sems[q]).wait()

                g2 = g + 2

                @pl.when((lax.rem(g2, _GRP) == 0) & (g2 < _NCHUNK))
                def _():
                    grp2 = lax.rem(lax.div(g2, _GRP), 2)
                    g2m = pl.multiple_of(g2, _GRP)
                    pltpu.sync_copy(src_hbm.at[s, pl.ds(g2m, _GRP)],
                                    srcg.at[grp2])
                    pltpu.sync_copy(dst_hbm.at[s, pl.ds(g2m, _GRP)],
                                    dstg.at[grp2])

                @pl.when(g2 < _NCHUNK)
                def _():
                    issue_gather(g2, q)
            return 0
        lax.fori_loop(0, (_NCHUNK + 2 + _NBUF - 1) // _NBUF, round_, 0)
        plsc.subcore_barrier()

        def dump(k2, _):
            r0 = s * _NPT + k2 * 2 * _RD
            ws = []
            for h in range(2):
                pltpu.sync_copy(acc.at[pl.ds(r0 + h * _RD, _RD)], dbuf)

                def drow(i, _):
                    for j in range(_HALF // 16):
                        d = dbuf[i, pl.ds(j * 16, 16)]
                        nm = dbuf[i, pl.ds(_HALF + j * 16, 16)]
                        abuf[h, i, pl.ds(j * 16, 16)] = nm / (d + 1e-16)
                    return 0
                lax.fori_loop(0, _RD, drow, 0)
                ws.append(pltpu.async_copy(
                    abuf.at[h], out_hbm.at[c, pl.ds(r0 + h * _RD, _RD)],
                    wsem))
            for cp in ws:
                cp.wait()
            return 0
        lax.fori_loop(0, _NPT // (2 * _RD), dump, 0)

    return k(t, src_r, dst_r, cm)


def kernel(x, edge_index, W_enc, b_enc, Wg, bg, gamma, beta, W_pred, b_pred):
    pad3 = ((0, 0), (0, _IPADT - _IROWT), (0, 0))
    src_r = jnp.pad(edge_index[0].reshape(16, _IROWT, _B), pad3)
    dst_r = jnp.pad(edge_index[1].reshape(16, _IROWT, _B), pad3)
    b_enc2 = b_enc.reshape(1, _D)
    wp = jnp.zeros((_D, _D), jnp.float32).at[:, :_C].set(W_pred)
    bp = jnp.zeros((1, _D), jnp.float32).at[0, :_C].set(b_pred)

    t, cm = _tc_encode(x, W_enc, b_enc2)
    h = None
    for l in range(_L):
        aggr2 = _sc_edge_pass(t, src_r, dst_r, cm.reshape(_D))
        h, t, cm = _tc_post(t, aggr2, Wg[l], bg[l].reshape(1, _D),
                            gamma[l].reshape(1, _D), beta[l].reshape(1, _D),
                            h)
    full = _tc_predict(t, wp, bp)
    return full[:, :_C]


# speedup vs baseline: 15.4566x; 1.1445x over previous
import functools

import jax
import jax.numpy as jnp
from jax import lax
from jax.experimental import pallas as pl
from jax.experimental.pallas import tpu as pltpu
from jax.experimental.pallas import tpu_sc as plsc

_N = 10000
_E = 320000
_D = 128
_C = 47
_L = 4

_BN = 2000
_B = 80
_NBUF = 4
_EPT = _E // 16
_NCHUNK = _EPT // _B
_GRP = 8
_IROWT = _NCHUNK
_IPADT = 256
_NP = 10240
_NPT = _NP // 16
_RD = 8
_HALF = _D // 2


def _tc_encode(x, w, b):
    def body(x_ref, w_ref, b_ref, h_ref, cm_ref):
        i = pl.program_id(0)
        h = jnp.dot(x_ref[...], w_ref[...],
                    preferred_element_type=jnp.float32,
                    precision=lax.Precision.HIGHEST) + b_ref[...]
        h_ref[...] = h
        m = jnp.max(jnp.maximum(h, 0.0), axis=0, keepdims=True) + 1e-7

        @pl.when(i == 0)
        def _():
            cm_ref[...] = m

        @pl.when(i > 0)
        def _():
            cm_ref[...] = jnp.maximum(cm_ref[...], m)

    return pl.pallas_call(
        body,
        grid=(_N // _BN,),
        in_specs=[
            pl.BlockSpec((_BN, _D), lambda i: (i, 0)),
            pl.BlockSpec((_D, _D), lambda i: (0, 0)),
            pl.BlockSpec((1, _D), lambda i: (0, 0)),
        ],
        out_specs=[
            pl.BlockSpec((_BN, _D), lambda i: (i, 0)),
            pl.BlockSpec((1, _D), lambda i: (0, 0)),
        ],
        out_shape=[
            jax.ShapeDtypeStruct((_N, _D), jnp.float32),
            jax.ShapeDtypeStruct((1, _D), jnp.float32),
        ],
    )(x, w, b)


def _tc_post(t, aggr2, w, b, g, be, hprev):
    residual = hprev is not None

    def body(*refs):
        if residual:
            t_ref, a_ref, w_ref, b_ref, g_ref, be_ref, hp_ref = refs[:7]
            h_ref, t2_ref, cm_ref = refs[7:]
        else:
            t_ref, a_ref, w_ref, b_ref, g_ref, be_ref = refs[:6]
            h_ref, t2_ref, cm_ref = refs[6:]
        i = pl.program_id(0)
        w_full = w_ref[...]
        a0 = a_ref[0]
        a1 = a_ref[1]
        dot = functools.partial(jnp.dot,
                                preferred_element_type=jnp.float32,
                                precision=lax.Precision.HIGHEST)
        hn = (dot(t_ref[...], w_full)
              + dot(a0, w_full[:_HALF, :])
              + dot(a1, w_full[_HALF:, :])
              + b_ref[...])
        if residual:
            hn = hn + hp_ref[...]
        h_ref[...] = hn
        mu = jnp.mean(hn, axis=1, keepdims=True)
        var = jnp.mean((hn - mu) ** 2, axis=1, keepdims=True)
        ln = (hn - mu) * lax.rsqrt(var + 1e-5) * g_ref[...] + be_ref[...]
        t2 = jnp.maximum(ln, 0.0)
        t2_ref[...] = t2
        m = jnp.max(t2, axis=0, keepdims=True) + 1e-7

        @pl.when(i == 0)
        def _():
            cm_ref[...] = m

        @pl.when(i > 0)
        def _():
            cm_ref[...] = jnp.maximum(cm_ref[...], m)

    in_specs = [
        pl.BlockSpec((_BN, _D), lambda i: (i, 0)),
        pl.BlockSpec((2, _BN, _HALF), lambda i: (0, i, 0)),
        pl.BlockSpec((_D, _D), lambda i: (0, 0)),
        pl.BlockSpec((1, _D), lambda i: (0, 0)),
        pl.BlockSpec((1, _D), lambda i: (0, 0)),
        pl.BlockSpec((1, _D), lambda i: (0, 0)),
    ]
    args = [t, aggr2, w, b, g, be]
    if residual:
        in_specs.append(pl.BlockSpec((_BN, _D), lambda i: (i, 0)))
        args.append(hprev)
    return pl.pallas_call(
        body,
        grid=(_N // _BN,),
        in_specs=in_specs,
        out_specs=[
            pl.BlockSpec((_BN, _D), lambda i: (i, 0)),
            pl.BlockSpec((_BN, _D), lambda i: (i, 0)),
            pl.BlockSpec((1, _D), lambda i: (0, 0)),
        ],
        out_shape=[
            jax.ShapeDtypeStruct((_N, _D), jnp.float32),
            jax.ShapeDtypeStruct((_N, _D), jnp.float32),
            jax.ShapeDtypeStruct((1, _D), jnp.float32),
        ],
    )(*args)


def _tc_predict(t, wp, bp):
    def body(t_ref, w_ref, b_ref, o_ref):
        lg = jnp.dot(t_ref[...], w_ref[...],
                     preferred_element_type=jnp.float32,
                     precision=lax.Precision.HIGHEST) + b_ref[...]
        mask = lax.broadcasted_iota(jnp.int32, (_BN, _D), 1) < _C
        mx = jnp.max(jnp.where(mask, lg, -jnp.inf), axis=1, keepdims=True)
        se = jnp.sum(jnp.where(mask, jnp.exp(lg - mx), 0.0),
                     axis=1, keepdims=True)
        o_ref[...] = lg - mx - jnp.log(se)

    return pl.pallas_call(
        body,
        grid=(_N // _BN,),
        in_specs=[
            pl.BlockSpec((_BN, _D), lambda i: (i, 0)),
            pl.BlockSpec((_D, _D), lambda i: (0, 0)),
            pl.BlockSpec((1, _D), lambda i: (0, 0)),
        ],
        out_specs=pl.BlockSpec((_BN, _D), lambda i: (i, 0)),
        out_shape=jax.ShapeDtypeStruct((_N, _D), jnp.float32),
    )(t, wp, bp)


def _sc_edge_pass(t, src_r, dst_r, cm):
    mesh = plsc.VectorSubcoreMesh(core_axis_name="c", subcore_axis_name="s")

    @functools.partial(
        pl.kernel,
        out_type=jax.ShapeDtypeStruct((2, _NP, _HALF), jnp.float32),
        mesh=mesh,
        scratch_types=[
            pltpu.VMEM((2, _GRP, _B), jnp.int32),
            pltpu.VMEM((2, _GRP, _B), jnp.int32),
            pltpu.VMEM((_NBUF, _B, _D), jnp.float32),
            pltpu.VMEM((_HALF,), jnp.float32),
            pltpu.VMEM((_RD, _D), jnp.float32),
            pltpu.VMEM((2, _RD, _HALF), jnp.float32),
            pltpu.VMEM_SHARED((_NP, _D), jnp.float32),
            pltpu.SemaphoreType.DMA,
            pltpu.SemaphoreType.DMA,
            pltpu.SemaphoreType.DMA,
            pltpu.SemaphoreType.DMA,
            pltpu.SemaphoreType.DMA,
        ],
    )
    def k(t_hbm, src_hbm, dst_hbm, cm_hbm, out_hbm,
          srcg, dstg, rows_v, cm_v, dbuf, abuf, acc, sm0, sm1, sm2, sm3,
          wsem):
        sems = [sm0, sm1, sm2, sm3]
        c = lax.axis_index("c")
        s = lax.axis_index("s")
        c64 = c * _HALF
        zv = jnp.zeros((16,), jnp.float32)

        def zrow(i, _):
            for j in range(_D // 16):
                rows_v[0, i, pl.ds(j * 16, 16)] = zv
            return 0
        lax.fori_loop(0, _B, zrow, 0)
        for z in range(_NPT // _B):
            pltpu.sync_copy(rows_v.at[0],
                            acc.at[pl.ds(s * _NPT + z * _B, _B)])
        pltpu.sync_copy(cm_hbm.at[pl.ds(c64, _HALF)], cm_v)
        plsc.subcore_barrier()

        cms = [cm_v[pl.ds(j * 16, 16)] for j in range(_HALF // 16)]
        nj = _HALF // 16

        def compute(p):
            def load_row(ii):
                return [rows_v[p, ii, pl.ds(c64 + j * 16, 16)]
                        for j in range(nj)]

            def store_row(ii, rs):
                ms = [jnp.maximum(r, 0.0) + 1e-7 for r in rs]
                es = [jnp.exp(ms[j] - cms[j]) for j in range(nj)]
                for j in range(nj):
                    rows_v[p, ii, pl.ds(j * 16, 16)] = es[j]
                    rows_v[p, ii, pl.ds(_HALF + j * 16, 16)] = ms[j] * es[j]

            def body(i, rs):
                rs_next = load_row(i + 1)
                store_row(i, rs)
                return rs_next
            last = lax.fori_loop(0, _B - 1, body, load_row(0))
            store_row(_B - 1, last)

        def issue_gather(g, p):
            grp = lax.rem(lax.div(g, _GRP), 2)
            row = lax.rem(g, _GRP)
            return pltpu.async_copy(t_hbm.at[srcg.at[grp, row]],
                                    rows_v.at[p], sems[p])

        pltpu.sync_copy(src_hbm.at[s, pl.ds(0, _GRP)], srcg.at[0])
        pltpu.sync_copy(dst_hbm.at[s, pl.ds(0, _GRP)], dstg.at[0])
        issue_gather(0, 0)
        issue_gather(1, 1)

        def round_(r, _):
            for p in range(_NBUF):
                g = r * _NBUF + p
                q = (p + 2) % _NBUF

                g2 = g + 2

                @pl.when(g < _NCHUNK)
                def _():
                    pltpu.make_async_copy(t_hbm.at[srcg.at[0, 0]],
                                          rows_v.at[p], sems[p]).wait()

                @pl.when((g >= 2) & (g < _NCHUNK + 2))
                def _():
                    pltpu.make_async_copy(rows_v.at[q],
                                          acc.at[dstg.at[0, 0]],
                                          sems[q]).wait()

                @pl.when((lax.rem(g2, _GRP) == 0) & (g2 < _NCHUNK))
                def _():
                    grp2 = lax.rem(lax.div(g2, _GRP), 2)
                    g2m = pl.multiple_of(g2, _GRP)
                    pltpu.sync_copy(src_hbm.at[s, pl.ds(g2m, _GRP)],
                                    srcg.at[grp2])
                    pltpu.sync_copy(dst_hbm.at[s, pl.ds(g2m, _GRP)],
                                    dstg.at[grp2])

                @pl.when(g2 < _NCHUNK)
                def _():
                    issue_gather(g2, q)

                @pl.when(g < _NCHUNK)
                def _():
                    compute(p)
                    grp = lax.rem(lax.div(g, _GRP), 2)
                    row = lax.rem(g, _GRP)
                    pltpu.async_copy(rows_v.at[p],
                                     acc.at[dstg.at[grp, row]],
                                     sems[p], add=True)
            return 0
        lax.fori_loop(0, (_NCHUNK + 2 + _NBUF - 1) // _NBUF, round_, 0)
        plsc.subcore_barrier()

        def dump(k2, _):
            r0 = s * _NPT + k2 * 2 * _RD
            ws = []
            for h in range(2):
                pltpu.sync_copy(acc.at[pl.ds(r0 + h * _RD, _RD)], dbuf)

                def drow(i, _):
                    for j in range(_HALF // 16):
                        d = dbuf[i, pl.ds(j * 16, 16)]
                        nm = dbuf[i, pl.ds(_HALF + j * 16, 16)]
                        abuf[h, i, pl.ds(j * 16, 16)] = nm / (d + 1e-16)
                    return 0
                lax.fori_loop(0, _RD, drow, 0)
                ws.append(pltpu.async_copy(
                    abuf.at[h], out_hbm.at[c, pl.ds(r0 + h * _RD, _RD)],
                    wsem))
            for cp in ws:
                cp.wait()
            return 0
        lax.fori_loop(0, _NPT // (2 * _RD), dump, 0)

    return k(t, src_r, dst_r, cm)


def kernel(x, edge_index, W_enc, b_enc, Wg, bg, gamma, beta, W_pred, b_pred):
    pad3 = ((0, 0), (0, _IPADT - _IROWT), (0, 0))
    src_r = jnp.pad(edge_index[0].reshape(16, _IROWT, _B), pad3)
    dst_r = jnp.pad(edge_index[1].reshape(16, _IROWT, _B), pad3)
    b_enc2 = b_enc.reshape(1, _D)
    wp = jnp.zeros((_D, _D), jnp.float32).at[:, :_C].set(W_pred)
    bp = jnp.zeros((1, _D), jnp.float32).at[0, :_C].set(b_pred)

    t, cm = _tc_encode(x, W_enc, b_enc2)
    h = None
    for l in range(_L):
        aggr2 = _sc_edge_pass(t, src_r, dst_r, cm.reshape(_D))
        h, t, cm = _tc_post(t, aggr2, Wg[l], bg[l].reshape(1, _D),
                            gamma[l].reshape(1, _D), beta[l].reshape(1, _D),
                            h)
    full = _tc_predict(t, wp, bp)
    return full[:, :_C]
